# baseline (speedup 1.0000x reference)
import jax
import jax.numpy as jnp
from jax.experimental import pallas as pl

P = 2
NE = 1024
ED = 256
BETA = 0.25

M = 512


def _vq_kernel(x_ref, wpe_ref, bpe_ref, g_ref, b_ref, embT_ref, emb_ref,
               wpu_ref, bpu_ref, out_ref, idx_ref, loss_ref):
    i = pl.program_id(0)
    n = pl.num_programs(0)
    x = x_ref[...]
    zp = jnp.dot(x, wpe_ref[...], preferred_element_type=jnp.float32) + bpe_ref[...]
    mu = jnp.mean(zp, axis=1, keepdims=True)
    zc = zp - mu
    var = jnp.mean(zc * zc, axis=1, keepdims=True)
    zp = zc / jnp.sqrt(var + 1e-5) * g_ref[...] + b_ref[...]

    embT = embT_ref[...]
    emb = emb_ref[...]
    esq = jnp.sum(emb * emb, axis=1)[None, :]
    rsq = jnp.sum(zp * zp, axis=1, keepdims=True)
    scores = jnp.dot(zp, embT, preferred_element_type=jnp.float32)
    dist = rsq + esq - 2.0 * scores
    minv = jnp.min(dist, axis=1, keepdims=True)
    cols = jax.lax.broadcasted_iota(jnp.int32, dist.shape, 1)
    idx = jnp.min(jnp.where(dist == minv, cols, NE), axis=1)
    onehot = (cols == idx[:, None]).astype(jnp.float32)
    zq = jnp.dot(onehot, emb, preferred_element_type=jnp.float32)

    diff = zq - zp
    s = jnp.sum(diff * diff)

    s2 = s.reshape(1, 1)

    @pl.when(i == 0)
    def _():
        loss_ref[...] = s2

    @pl.when(i != 0)
    def _():
        loss_ref[...] = loss_ref[...] + s2

    @pl.when(i == n - 1)
    def _():
        loss_ref[...] = loss_ref[...] * ((1.0 + BETA) / (n * M * ED))

    out_ref[...] = jnp.dot(zq, wpu_ref[...], preferred_element_type=jnp.float32) + bpu_ref[...]
    idx_ref[0, 0, :] = idx


def kernel(z, emb, W_pe, b_pe, gamma, beta_ln, W_pu, b_pu):
    b, c, h, w = z.shape
    hp, wp = h // P, w // P
    D = c * P * P
    patches = z.reshape(b, c, hp, P, wp, P).transpose(0, 2, 4, 1, 3, 5).reshape(b * hp * wp, D)
    N = patches.shape[0]
    grid = N // M

    out_p, idx3, loss = pl.pallas_call(
        _vq_kernel,
        grid=(grid,),
        in_specs=[
            pl.BlockSpec((M, D), lambda i: (i, 0)),
            pl.BlockSpec((D, ED), lambda i: (0, 0)),
            pl.BlockSpec((1, ED), lambda i: (0, 0)),
            pl.BlockSpec((1, ED), lambda i: (0, 0)),
            pl.BlockSpec((1, ED), lambda i: (0, 0)),
            pl.BlockSpec((ED, NE), lambda i: (0, 0)),
            pl.BlockSpec((NE, ED), lambda i: (0, 0)),
            pl.BlockSpec((ED, D), lambda i: (0, 0)),
            pl.BlockSpec((1, D), lambda i: (0, 0)),
        ],
        out_specs=[
            pl.BlockSpec((M, D), lambda i: (i, 0)),
            pl.BlockSpec((1, 1, M), lambda i: (i, 0, 0)),
            pl.BlockSpec((1, 1), lambda i: (0, 0)),
        ],
        out_shape=[
            jax.ShapeDtypeStruct((N, D), jnp.float32),
            jax.ShapeDtypeStruct((grid, 1, M), jnp.int32),
            jax.ShapeDtypeStruct((1, 1), jnp.float32),
        ],
    )(patches, W_pe.T, b_pe.reshape(1, ED), gamma.reshape(1, ED),
      beta_ln.reshape(1, ED), emb.T, emb, W_pu.T, b_pu.reshape(1, D))

    idx = idx3.reshape(N)
    out = out_p.reshape(b, hp, wp, c, P, P).transpose(0, 3, 1, 4, 2, 5).reshape(b, c, h, w)
    return out, loss[0, 0], idx
